# baseline (speedup 1.0000x reference)
import dataclasses
import functools

import jax
import jax.numpy as jnp
from jax import lax
from jax.experimental import pallas as pl
from jax.experimental.pallas import tpu as pltpu
from jax.experimental.pallas import tpu_sc as plsc

F32 = jnp.float32
I32 = jnp.int32

NC = 2
NS = 16
NW = NC * NS
LANES = 16
EB = 128

_MESH = plsc.VectorSubcoreMesh(
    core_axis_name="c", subcore_axis_name="s", num_cores=NC, num_subcores=NS
)

_SC_PARAMS = pltpu.CompilerParams()
if "needs_layout_passes" in pltpu.CompilerParams.__dataclass_fields__:
    _SC_PARAMS = dataclasses.replace(_SC_PARAMS, needs_layout_passes=False)
if "internal_scratch_in_bytes" in pltpu.CompilerParams.__dataclass_fields__:
    _SC_PARAMS = dataclasses.replace(_SC_PARAMS, internal_scratch_in_bytes=4096)
_SC_PARAMS_LINEAR = dataclasses.replace(_SC_PARAMS, use_tc_tiling_on_sc=False)


def _wid(cid, sid):
    return sid * NC + cid


def _bcast16(v):
    return jnp.full((LANES,), v, I32)


def _deg_sc(dst, w, n_pad, nb):
    @functools.partial(
        pl.kernel,
        out_type=jax.ShapeDtypeStruct((NC, NS, n_pad), F32),
        mesh=_MESH,
        compiler_params=_SC_PARAMS,
        scratch_types=[
            pltpu.VMEM((nb, EB), I32),
            pltpu.VMEM((nb, EB), F32),
            pltpu.VMEM((n_pad,), F32),
            pltpu.SemaphoreType.DMA,
        ],
    )
    def k(dst_hbm, w_hbm, out_hbm, dst_v, w_v, deg_v, sem):
        cid = lax.axis_index("c")
        sid = lax.axis_index("s")
        zeros = jnp.zeros((LANES,), F32)

        blk0 = sid * (2 * nb) + cid * nb
        cp1 = pltpu.async_copy(dst_hbm.at[pl.ds(blk0, nb)], dst_v, sem)
        cp2 = pltpu.async_copy(w_hbm.at[pl.ds(blk0, nb)], w_v, sem)

        @pl.loop(0, n_pad // LANES, unroll=4)
        def _(i):
            deg_v[pl.ds(i * LANES, LANES)] = zeros

        cp1.wait()
        cp2.wait()

        @pl.loop(0, nb)
        def _(b):
            for j in range(EB // LANES):
                sl = pl.ds(j * LANES, LANES)
                plsc.addupdate_scatter(deg_v, [dst_v[b, sl]], w_v[b, sl])

        pltpu.sync_copy(deg_v, out_hbm.at[cid, sid])

    return k(dst, w)


def _spmm_sc(src, dst, w, dinv, psp, h, n_pad, nbw, split):
    stripe = n_pad // NS
    passes = split // 2
    hh = h // split
    hl = hh // LANES

    @functools.partial(
        pl.kernel,
        out_type=jax.ShapeDtypeStruct((NC, passes, n_pad, hh), F32),
        mesh=_MESH,
        compiler_params=_SC_PARAMS_LINEAR,
        scratch_types=[
            pltpu.VMEM((nbw, EB), I32),
            pltpu.VMEM((nbw, EB), I32),
            pltpu.VMEM((nbw, EB), F32),
            pltpu.VMEM((nbw * EB,), F32),
            pltpu.VMEM((n_pad,), F32),
            pltpu.VMEM((EB, hh), F32),
            pltpu.VMEM((EB, hh), F32),
            pltpu.VMEM((EB, hh), F32),
            pltpu.VMEM_SHARED((n_pad, hh), F32),
            pltpu.SemaphoreType.DMA,
            pltpu.SemaphoreType.DMA,
            pltpu.SemaphoreType.DMA,
        ],
    )
    def k(src_hbm, dst_hbm, w_hbm, dinv_hbm, psp_hbm, out_hbm,
          src_v, dst_v, w_v, norm_v, dinv_v, rows0, rows1, zbuf, acc_sh,
          sem_in, gsem, ssem):
        cid = lax.axis_index("c")
        sid = lax.axis_index("s")
        zeros = jnp.zeros((LANES,), F32)

        cps = [
            pltpu.async_copy(src_hbm.at[pl.ds(sid * nbw, nbw)], src_v, sem_in),
            pltpu.async_copy(dst_hbm.at[pl.ds(sid * nbw, nbw)], dst_v, sem_in),
            pltpu.async_copy(w_hbm.at[pl.ds(sid * nbw, nbw)], w_v, sem_in),
            pltpu.async_copy(dinv_hbm, dinv_v, sem_in),
        ]

        @pl.loop(0, EB, unroll=2)
        def _(r):
            for c in range(hl):
                zbuf[r, pl.ds(c * LANES, LANES)] = zeros

        row0 = sid * stripe

        def zero_stripe():
            for i in range(stripe // EB):
                pltpu.sync_copy(zbuf, acc_sh.at[pl.ds(row0 + i * EB, EB)])

        zero_stripe()
        for cp in cps:
            cp.wait()

        @pl.loop(0, nbw)
        def _(b):
            for j in range(EB // LANES):
                sl = pl.ds(j * LANES, LANES)
                s16 = src_v[b, sl]
                nrm = (plsc.load_gather(dinv_v, [s16]) * w_v[b, sl]
                       * plsc.load_gather(dinv_v, [dst_v[b, sl]]))
                norm_v[pl.ds(b * EB + j * LANES, LANES)] = nrm
                src_v[b, sl] = s16 * split + passes * cid

        plsc.subcore_barrier()

        def step(b, cur, oth, first, last):
            def wait_prev():
                pltpu.make_async_copy(
                    oth, acc_sh.at[dst_v.at[b - 1]], ssem).wait()

            if first is None:
                wait_prev()
            else:
                pl.when(first)(wait_prev)

            def fetch_next():
                pltpu.async_copy(psp_hbm.at[src_v.at[b + 1]], oth, gsem)

            if last is None:
                fetch_next()
            else:
                pl.when(last)(fetch_next)

            pltpu.make_async_copy(psp_hbm.at[src_v.at[b]], cur, gsem).wait()

            @pl.loop(0, EB // LANES)
            def _(g):
                k16 = jnp.arange(LANES, dtype=I32) + g * LANES
                ng = norm_v[pl.ds(b * EB + g * LANES, LANES)]
                vs = [
                    plsc.load_gather(cur, [k16, jnp.full((LANES,), f, I32)])
                    for f in range(hh)
                ]
                for f in range(hh):
                    plsc.store_scatter(
                        cur, [k16, jnp.full((LANES,), f, I32)], vs[f] * ng)

            pltpu.async_copy(cur, acc_sh.at[dst_v.at[b]], ssem, add=True)

        for ps in range(passes):
            pltpu.async_copy(psp_hbm.at[src_v.at[0]], rows0, gsem)

            @pl.loop(0, nbw // 2)
            def _(ob):
                step(ob * 2, rows0, rows1, first=ob > 0, last=None)
                step(ob * 2 + 1, rows1, rows0,
                     first=None, last=ob < nbw // 2 - 1)

            pltpu.make_async_copy(
                rows1, acc_sh.at[dst_v.at[nbw - 1]], ssem).wait()
            plsc.subcore_barrier()
            pltpu.sync_copy(
                acc_sh.at[pl.ds(row0, stripe)],
                out_hbm.at[cid, ps, pl.ds(row0, stripe)],
            )
            if ps < passes - 1:
                zero_stripe()

                @pl.loop(0, nbw)
                def _(b):
                    for j in range(EB // LANES):
                        sl = pl.ds(j * LANES, LANES)
                        src_v[b, sl] = src_v[b, sl] + 1

                plsc.subcore_barrier()

    return k(src, dst, w, dinv, psp)


def _pool_sc(h3, posf, s, m, h):
    nb = posf.shape[0] // NW
    per_w = nb * EB
    gb = EB // m
    hl = h // LANES

    @functools.partial(
        pl.kernel,
        out_type=jax.ShapeDtypeStruct((s, h), F32),
        mesh=_MESH,
        compiler_params=_SC_PARAMS,
        scratch_types=[
            pltpu.VMEM((nb, EB), I32),
            pltpu.VMEM((EB, h), F32),
            pltpu.VMEM((EB, h), F32),
            pltpu.VMEM((2 * gb, h), F32),
            pltpu.SemaphoreType.DMA,
            pltpu.SemaphoreType.DMA,
        ],
    )
    def k(h3_hbm, pos_hbm, out_hbm, idx_v, rows0, rows1, emb_v, sem, gsem):
        cid = lax.axis_index("c")
        sid = lax.axis_index("s")
        wid = _wid(cid, sid)

        pltpu.async_copy(
            pos_hbm.at[pl.ds(wid * nb, nb)], idx_v, sem).wait()
        pltpu.async_copy(h3_hbm.at[idx_v.at[0]], rows0, gsem)

        def reduce_block(cur, half):
            for g in range(gb):
                for c in range(hl):
                    sl = pl.ds(c * LANES, LANES)
                    acc = cur[g * m, sl]
                    for r in range(1, m):
                        acc = acc + cur[g * m + r, sl]
                    emb_v[half * gb + g, sl] = acc

        @pl.loop(0, nb // 2)
        def _(ob):
            b = ob * 2
            pltpu.async_copy(h3_hbm.at[idx_v.at[b + 1]], rows1, gsem)
            pltpu.make_async_copy(h3_hbm.at[idx_v.at[b]], rows0, gsem).wait()
            reduce_block(rows0, 0)

            @pl.when(ob < nb // 2 - 1)
            def _():
                pltpu.async_copy(h3_hbm.at[idx_v.at[b + 2]], rows0, gsem)

            pltpu.make_async_copy(
                h3_hbm.at[idx_v.at[b + 1]], rows1, gsem).wait()
            reduce_block(rows1, 1)
            pltpu.sync_copy(
                emb_v,
                out_hbm.at[pl.ds(wid * (per_w // m) + ob * 2 * gb, 2 * gb)])

    return k(h3, posf)


def _matmul_tc(x, w, blk):
    n, d = x.shape
    h = w.shape[1]

    def body(x_ref, w_ref, o_ref):
        o_ref[...] = jnp.dot(x_ref[...], w_ref[...],
                             preferred_element_type=F32)

    return pl.pallas_call(
        body,
        grid=(n // blk,),
        in_specs=[
            pl.BlockSpec((blk, d), lambda i: (i, 0)),
            pl.BlockSpec((d, h), lambda i: (0, 0)),
        ],
        out_specs=pl.BlockSpec((blk, h), lambda i: (i, 0)),
        out_shape=jax.ShapeDtypeStruct((n, h), F32),
    )(x, w)


def _dinv_tc(deg32):
    nw, n_pad = deg32.shape

    def body(d_ref, o_ref):
        deg = jnp.sum(d_ref[...], axis=0) + 1.0
        o_ref[0, :] = jnp.where(deg > 0, lax.rsqrt(deg), 0.0)

    return pl.pallas_call(
        body,
        out_shape=jax.ShapeDtypeStruct((1, n_pad), F32),
    )(deg32)


def _mid_tc(acc, p, dinv_col, bias, g, be, rm, rv, w_next, blk, split):
    n, h = p.shape
    passes = split // 2
    have_mm = w_next is not None

    def body(*refs):
        if have_mm:
            (a_ref, p_ref, di_ref, b_ref, g_ref, be_ref, rm_ref, rv_ref,
             w_ref, o_ref) = refs
        else:
            (a_ref, p_ref, di_ref, b_ref, g_ref, be_ref, rm_ref, rv_ref,
             o_ref) = refs
        di = di_ref[...]
        agg = jnp.concatenate(
            [a_ref[c, ps] for c in range(NC) for ps in range(passes)],
            axis=1)
        z = (agg + di * di * p_ref[...]) + b_ref[...]
        z = (z - rm_ref[...]) * lax.rsqrt(rv_ref[...] + 1e-5) * g_ref[...] + be_ref[...]
        z = jnp.maximum(z, 0.0)
        if have_mm:
            z = jnp.dot(z, w_ref[...], preferred_element_type=F32)
        o_ref[...] = z

    in_specs = [
        pl.BlockSpec((NC, passes, blk, h // split), lambda i: (0, 0, i, 0)),
        pl.BlockSpec((blk, h), lambda i: (i, 0)),
        pl.BlockSpec((blk, 1), lambda i: (i, 0)),
        pl.BlockSpec((1, h), lambda i: (0, 0)),
        pl.BlockSpec((1, h), lambda i: (0, 0)),
        pl.BlockSpec((1, h), lambda i: (0, 0)),
        pl.BlockSpec((1, h), lambda i: (0, 0)),
        pl.BlockSpec((1, h), lambda i: (0, 0)),
    ]
    args = [acc, p, dinv_col, bias, g, be, rm, rv]
    if have_mm:
        in_specs.append(pl.BlockSpec((h, h), lambda i: (0, 0)))
        args.append(w_next)

    return pl.pallas_call(
        body,
        grid=(n // blk,),
        in_specs=in_specs,
        out_specs=pl.BlockSpec((blk, h), lambda i: (i, 0)),
        out_shape=jax.ShapeDtypeStruct((n, h), F32),
    )(*args)


def _mlp_tc(emb_sum, inv_m, wc1, bc1, wc2, bc2, blk):
    s, h = emb_sum.shape
    h2 = wc1.shape[1]
    c = wc2.shape[1]

    def body(e_ref, w1_ref, b1_ref, w2_ref, b2_ref, o_ref):
        e = e_ref[...] * inv_m
        z = jnp.dot(e, w1_ref[...], preferred_element_type=F32) + b1_ref[...]
        z = jnp.maximum(z, 0.0)
        o_ref[...] = jnp.dot(z, w2_ref[...],
                             preferred_element_type=F32) + b2_ref[...]

    return pl.pallas_call(
        body,
        grid=(s // blk,),
        in_specs=[
            pl.BlockSpec((blk, h), lambda i: (i, 0)),
            pl.BlockSpec((h, h2), lambda i: (0, 0)),
            pl.BlockSpec((1, h2), lambda i: (0, 0)),
            pl.BlockSpec((h2, c), lambda i: (0, 0)),
            pl.BlockSpec((1, c), lambda i: (0, 0)),
        ],
        out_specs=pl.BlockSpec((blk, c), lambda i: (i, 0)),
        out_shape=jax.ShapeDtypeStruct((s, c), F32),
    )(emb_sum, wc1, bc1, wc2, bc2)


def kernel(x, edge_index, edge_attr, pos, params):
    n, d = x.shape
    e = edge_index.shape[1]
    s, m = pos.shape
    h = params["W0"].shape[1]

    nbw = -(-e // (NS * EB))
    nbw = -(-nbw // 16) * 16
    nb = nbw // 2
    e_pad = nbw * NS * EB
    pad = e_pad - e
    src = edge_index[0]
    dst = edge_index[1]
    w = edge_attr.astype(F32)
    if pad:
        fill = (jnp.arange(pad, dtype=I32) * 97) % n
        src = jnp.concatenate([src, fill])
        dst = jnp.concatenate([dst, fill])
        w = jnp.concatenate([w, jnp.zeros((pad,), F32)])
    src = src.reshape(NS * nbw, EB)
    dst = dst.reshape(NS * nbw, EB)
    w = w.reshape(NS * nbw, EB)

    n_pad = -(-n // (NS * EB)) * (NS * EB)
    x_p = jnp.zeros((n_pad, d), F32).at[:n].set(x)

    deg32 = _deg_sc(dst, w, n_pad, nb).reshape(NW, n_pad)
    dinv_row = _dinv_tc(deg32)
    dinv = dinv_row.reshape(n_pad)
    dinv_col = dinv.reshape(n_pad, 1)

    blk = 1024
    p0 = _matmul_tc(x_p, params["W0"], blk)

    def layer(p_cur, i, w_next, split):
        psp = p_cur.reshape(split * n_pad, h // split)
        acc = _spmm_sc(src, dst, w, dinv, psp, h, n_pad, nbw, split)
        return _mid_tc(
            acc, p_cur, dinv_col,
            params["b%d" % i].reshape(1, h),
            params["g%d" % i].reshape(1, h),
            params["be%d" % i].reshape(1, h),
            params["rm%d" % i].reshape(1, h),
            params["rv%d" % i].reshape(1, h),
            w_next, blk, split,
        )

    p1 = layer(p0, 0, params["W1"], 4)
    p2 = layer(p1, 1, params["W2"], 4)
    h3 = layer(p2, 2, None, 4)

    posf = pos.reshape(NW * (s * m // (NW * EB)), EB).astype(I32)
    emb_sum = _pool_sc(h3, posf, s, m, h)

    h2 = params["Wc1"].shape[1]
    c = params["Wc2"].shape[1]
    return _mlp_tc(
        emb_sum, 1.0 / m,
        params["Wc1"], params["bc1"].reshape(1, h2),
        params["Wc2"], params["bc2"].reshape(1, c),
        blk=1024,
    )


# speedup vs baseline: 4.3756x; 4.3756x over previous
import dataclasses
import functools

import jax
import jax.numpy as jnp
from jax import lax
from jax.experimental import pallas as pl
from jax.experimental.pallas import tpu as pltpu
from jax.experimental.pallas import tpu_sc as plsc

F32 = jnp.float32
I32 = jnp.int32

NC = 2
NS = 16
NW = NC * NS
LANES = 16
EB = 128

_MESH = plsc.VectorSubcoreMesh(
    core_axis_name="c", subcore_axis_name="s", num_cores=NC, num_subcores=NS
)

_SC_PARAMS = pltpu.CompilerParams()
if "needs_layout_passes" in pltpu.CompilerParams.__dataclass_fields__:
    _SC_PARAMS = dataclasses.replace(_SC_PARAMS, needs_layout_passes=False)
if "internal_scratch_in_bytes" in pltpu.CompilerParams.__dataclass_fields__:
    _SC_PARAMS = dataclasses.replace(_SC_PARAMS, internal_scratch_in_bytes=4096)
_SC_PARAMS_LINEAR = dataclasses.replace(_SC_PARAMS, use_tc_tiling_on_sc=False)


def _wid(cid, sid):
    return sid * NC + cid


def _bcast16(v):
    return jnp.full((LANES,), v, I32)


_GDN = lax.GatherDimensionNumbers(
    offset_dims=(), collapsed_slice_dims=(0,), start_index_map=(0,))


def _lane_bcast(vec, j):
    idx = jnp.full((LANES, 1), j, I32)
    return lax.gather(vec, idx, _GDN, (1,),
                      mode=lax.GatherScatterMode.PROMISE_IN_BOUNDS)


def _deg_sc(dst, w, n_pad, nb):
    @functools.partial(
        pl.kernel,
        out_type=jax.ShapeDtypeStruct((NC, NS, n_pad), F32),
        mesh=_MESH,
        compiler_params=_SC_PARAMS,
        scratch_types=[
            pltpu.VMEM((nb, EB), I32),
            pltpu.VMEM((nb, EB), F32),
            pltpu.VMEM((n_pad,), F32),
            pltpu.SemaphoreType.DMA,
        ],
    )
    def k(dst_hbm, w_hbm, out_hbm, dst_v, w_v, deg_v, sem):
        cid = lax.axis_index("c")
        sid = lax.axis_index("s")
        zeros = jnp.zeros((LANES,), F32)

        blk0 = sid * (2 * nb) + cid * nb
        cp1 = pltpu.async_copy(dst_hbm.at[pl.ds(blk0, nb)], dst_v, sem)
        cp2 = pltpu.async_copy(w_hbm.at[pl.ds(blk0, nb)], w_v, sem)

        @pl.loop(0, n_pad // LANES, unroll=4)
        def _(i):
            deg_v[pl.ds(i * LANES, LANES)] = zeros

        cp1.wait()
        cp2.wait()

        @pl.loop(0, nb)
        def _(b):
            for j in range(EB // LANES):
                sl = pl.ds(j * LANES, LANES)
                plsc.addupdate_scatter(deg_v, [dst_v[b, sl]], w_v[b, sl])

        pltpu.sync_copy(deg_v, out_hbm.at[cid, sid])

    return k(dst, w)


def _spmm_sc(src, dst, w, dinv, psp, h, n_pad, nbw, split):
    stripe = n_pad // NS
    passes = split // 2
    hh = h // split
    hl = hh // LANES

    @functools.partial(
        pl.kernel,
        out_type=jax.ShapeDtypeStruct((NC, passes, n_pad, hh), F32),
        mesh=_MESH,
        compiler_params=_SC_PARAMS_LINEAR,
        scratch_types=[
            pltpu.VMEM((nbw, EB), I32),
            pltpu.VMEM((nbw, EB), I32),
            pltpu.VMEM((nbw, EB), F32),
            pltpu.VMEM((nbw * EB,), F32),
            pltpu.VMEM((n_pad,), F32),
            pltpu.VMEM((EB, hh), F32),
            pltpu.VMEM((EB, hh), F32),
            pltpu.VMEM((EB, hh), F32),
            pltpu.VMEM_SHARED((n_pad, hh), F32),
            pltpu.SemaphoreType.DMA,
            pltpu.SemaphoreType.DMA,
            pltpu.SemaphoreType.DMA,
        ],
    )
    def k(src_hbm, dst_hbm, w_hbm, dinv_hbm, psp_hbm, out_hbm,
          src_v, dst_v, w_v, norm_v, dinv_v, rows0, rows1, zbuf, acc_sh,
          sem_in, gsem, ssem):
        cid = lax.axis_index("c")
        sid = lax.axis_index("s")
        zeros = jnp.zeros((LANES,), F32)

        cps = [
            pltpu.async_copy(src_hbm.at[pl.ds(sid * nbw, nbw)], src_v, sem_in),
            pltpu.async_copy(dst_hbm.at[pl.ds(sid * nbw, nbw)], dst_v, sem_in),
            pltpu.async_copy(w_hbm.at[pl.ds(sid * nbw, nbw)], w_v, sem_in),
            pltpu.async_copy(dinv_hbm, dinv_v, sem_in),
        ]

        @pl.loop(0, EB, unroll=2)
        def _(r):
            for c in range(hl):
                zbuf[r, pl.ds(c * LANES, LANES)] = zeros

        row0 = sid * stripe

        def zero_stripe():
            for i in range(stripe // EB):
                pltpu.sync_copy(zbuf, acc_sh.at[pl.ds(row0 + i * EB, EB)])

        zero_stripe()
        for cp in cps:
            cp.wait()

        @pl.loop(0, nbw)
        def _(b):
            for j in range(EB // LANES):
                sl = pl.ds(j * LANES, LANES)
                s16 = src_v[b, sl]
                nrm = (plsc.load_gather(dinv_v, [s16]) * w_v[b, sl]
                       * plsc.load_gather(dinv_v, [dst_v[b, sl]]))
                norm_v[pl.ds(b * EB + j * LANES, LANES)] = nrm
                src_v[b, sl] = s16 * split + passes * cid

        plsc.subcore_barrier()

        def step(b, cur, oth, first, last):
            def wait_prev():
                pltpu.make_async_copy(
                    oth, acc_sh.at[dst_v.at[b - 1]], ssem).wait()

            if first is None:
                wait_prev()
            else:
                pl.when(first)(wait_prev)

            def fetch_next():
                pltpu.async_copy(psp_hbm.at[src_v.at[b + 1]], oth, gsem)

            if last is None:
                fetch_next()
            else:
                pl.when(last)(fetch_next)

            pltpu.make_async_copy(psp_hbm.at[src_v.at[b]], cur, gsem).wait()

            @pl.loop(0, EB // LANES)
            def _(g):
                ng = norm_v[pl.ds(b * EB + g * LANES, LANES)]
                for j in range(LANES):
                    bc = _lane_bcast(ng, j)
                    kk = g * LANES + j
                    for c in range(hl):
                        sl = pl.ds(c * LANES, LANES)
                        cur[kk, sl] = cur[kk, sl] * bc

            pltpu.async_copy(cur, acc_sh.at[dst_v.at[b]], ssem, add=True)

        for ps in range(passes):
            pltpu.async_copy(psp_hbm.at[src_v.at[0]], rows0, gsem)

            @pl.loop(0, nbw // 2)
            def _(ob):
                step(ob * 2, rows0, rows1, first=ob > 0, last=None)
                step(ob * 2 + 1, rows1, rows0,
                     first=None, last=ob < nbw // 2 - 1)

            pltpu.make_async_copy(
                rows1, acc_sh.at[dst_v.at[nbw - 1]], ssem).wait()
            plsc.subcore_barrier()
            pltpu.sync_copy(
                acc_sh.at[pl.ds(row0, stripe)],
                out_hbm.at[cid, ps, pl.ds(row0, stripe)],
            )
            if ps < passes - 1:
                zero_stripe()

                @pl.loop(0, nbw)
                def _(b):
                    for j in range(EB // LANES):
                        sl = pl.ds(j * LANES, LANES)
                        src_v[b, sl] = src_v[b, sl] + 1

                plsc.subcore_barrier()

    return k(src, dst, w, dinv, psp)


def _pool_sc(h3, posf, s, m, h):
    nb = posf.shape[0] // NW
    per_w = nb * EB
    gb = EB // m
    hl = h // LANES

    @functools.partial(
        pl.kernel,
        out_type=jax.ShapeDtypeStruct((s, h), F32),
        mesh=_MESH,
        compiler_params=_SC_PARAMS,
        scratch_types=[
            pltpu.VMEM((nb, EB), I32),
            pltpu.VMEM((EB, h), F32),
            pltpu.VMEM((EB, h), F32),
            pltpu.VMEM((2 * gb, h), F32),
            pltpu.SemaphoreType.DMA,
            pltpu.SemaphoreType.DMA,
        ],
    )
    def k(h3_hbm, pos_hbm, out_hbm, idx_v, rows0, rows1, emb_v, sem, gsem):
        cid = lax.axis_index("c")
        sid = lax.axis_index("s")
        wid = _wid(cid, sid)

        pltpu.async_copy(
            pos_hbm.at[pl.ds(wid * nb, nb)], idx_v, sem).wait()
        pltpu.async_copy(h3_hbm.at[idx_v.at[0]], rows0, gsem)

        def reduce_block(cur, half):
            for g in range(gb):
                for c in range(hl):
                    sl = pl.ds(c * LANES, LANES)
                    acc = cur[g * m, sl]
                    for r in range(1, m):
                        acc = acc + cur[g * m + r, sl]
                    emb_v[half * gb + g, sl] = acc

        @pl.loop(0, nb // 2)
        def _(ob):
            b = ob * 2
            pltpu.async_copy(h3_hbm.at[idx_v.at[b + 1]], rows1, gsem)
            pltpu.make_async_copy(h3_hbm.at[idx_v.at[b]], rows0, gsem).wait()
            reduce_block(rows0, 0)

            @pl.when(ob < nb // 2 - 1)
            def _():
                pltpu.async_copy(h3_hbm.at[idx_v.at[b + 2]], rows0, gsem)

            pltpu.make_async_copy(
                h3_hbm.at[idx_v.at[b + 1]], rows1, gsem).wait()
            reduce_block(rows1, 1)
            pltpu.sync_copy(
                emb_v,
                out_hbm.at[pl.ds(wid * (per_w // m) + ob * 2 * gb, 2 * gb)])

    return k(h3, posf)


def _matmul_tc(x, w, blk):
    n, d = x.shape
    h = w.shape[1]

    def body(x_ref, w_ref, o_ref):
        o_ref[...] = jnp.dot(x_ref[...], w_ref[...],
                             preferred_element_type=F32)

    return pl.pallas_call(
        body,
        grid=(n // blk,),
        in_specs=[
            pl.BlockSpec((blk, d), lambda i: (i, 0)),
            pl.BlockSpec((d, h), lambda i: (0, 0)),
        ],
        out_specs=pl.BlockSpec((blk, h), lambda i: (i, 0)),
        out_shape=jax.ShapeDtypeStruct((n, h), F32),
    )(x, w)


def _dinv_tc(deg32):
    nw, n_pad = deg32.shape

    def body(d_ref, o_ref):
        deg = jnp.sum(d_ref[...], axis=0) + 1.0
        o_ref[0, :] = jnp.where(deg > 0, lax.rsqrt(deg), 0.0)

    return pl.pallas_call(
        body,
        out_shape=jax.ShapeDtypeStruct((1, n_pad), F32),
    )(deg32)


def _mid_tc(acc, p, dinv_col, bias, g, be, rm, rv, w_next, blk, split):
    n, h = p.shape
    passes = split // 2
    have_mm = w_next is not None

    def body(*refs):
        if have_mm:
            (a_ref, p_ref, di_ref, b_ref, g_ref, be_ref, rm_ref, rv_ref,
             w_ref, o_ref) = refs
        else:
            (a_ref, p_ref, di_ref, b_ref, g_ref, be_ref, rm_ref, rv_ref,
             o_ref) = refs
        di = di_ref[...]
        agg = jnp.concatenate(
            [a_ref[c, ps] for c in range(NC) for ps in range(passes)],
            axis=1)
        z = (agg + di * di * p_ref[...]) + b_ref[...]
        z = (z - rm_ref[...]) * lax.rsqrt(rv_ref[...] + 1e-5) * g_ref[...] + be_ref[...]
        z = jnp.maximum(z, 0.0)
        if have_mm:
            z = jnp.dot(z, w_ref[...], preferred_element_type=F32)
        o_ref[...] = z

    in_specs = [
        pl.BlockSpec((NC, passes, blk, h // split), lambda i: (0, 0, i, 0)),
        pl.BlockSpec((blk, h), lambda i: (i, 0)),
        pl.BlockSpec((blk, 1), lambda i: (i, 0)),
        pl.BlockSpec((1, h), lambda i: (0, 0)),
        pl.BlockSpec((1, h), lambda i: (0, 0)),
        pl.BlockSpec((1, h), lambda i: (0, 0)),
        pl.BlockSpec((1, h), lambda i: (0, 0)),
        pl.BlockSpec((1, h), lambda i: (0, 0)),
    ]
    args = [acc, p, dinv_col, bias, g, be, rm, rv]
    if have_mm:
        in_specs.append(pl.BlockSpec((h, h), lambda i: (0, 0)))
        args.append(w_next)

    return pl.pallas_call(
        body,
        grid=(n // blk,),
        in_specs=in_specs,
        out_specs=pl.BlockSpec((blk, h), lambda i: (i, 0)),
        out_shape=jax.ShapeDtypeStruct((n, h), F32),
    )(*args)


def _mlp_tc(emb_sum, inv_m, wc1, bc1, wc2, bc2, blk):
    s, h = emb_sum.shape
    h2 = wc1.shape[1]
    c = wc2.shape[1]

    def body(e_ref, w1_ref, b1_ref, w2_ref, b2_ref, o_ref):
        e = e_ref[...] * inv_m
        z = jnp.dot(e, w1_ref[...], preferred_element_type=F32) + b1_ref[...]
        z = jnp.maximum(z, 0.0)
        o_ref[...] = jnp.dot(z, w2_ref[...],
                             preferred_element_type=F32) + b2_ref[...]

    return pl.pallas_call(
        body,
        grid=(s // blk,),
        in_specs=[
            pl.BlockSpec((blk, h), lambda i: (i, 0)),
            pl.BlockSpec((h, h2), lambda i: (0, 0)),
            pl.BlockSpec((1, h2), lambda i: (0, 0)),
            pl.BlockSpec((h2, c), lambda i: (0, 0)),
            pl.BlockSpec((1, c), lambda i: (0, 0)),
        ],
        out_specs=pl.BlockSpec((blk, c), lambda i: (i, 0)),
        out_shape=jax.ShapeDtypeStruct((s, c), F32),
    )(emb_sum, wc1, bc1, wc2, bc2)


def kernel(x, edge_index, edge_attr, pos, params):
    n, d = x.shape
    e = edge_index.shape[1]
    s, m = pos.shape
    h = params["W0"].shape[1]

    nbw = -(-e // (NS * EB))
    nbw = -(-nbw // 16) * 16
    nb = nbw // 2
    e_pad = nbw * NS * EB
    pad = e_pad - e
    src = edge_index[0]
    dst = edge_index[1]
    w = edge_attr.astype(F32)
    if pad:
        fill = (jnp.arange(pad, dtype=I32) * 97) % n
        src = jnp.concatenate([src, fill])
        dst = jnp.concatenate([dst, fill])
        w = jnp.concatenate([w, jnp.zeros((pad,), F32)])
    src = src.reshape(NS * nbw, EB)
    dst = dst.reshape(NS * nbw, EB)
    w = w.reshape(NS * nbw, EB)

    n_pad = -(-n // (NS * EB)) * (NS * EB)
    x_p = jnp.zeros((n_pad, d), F32).at[:n].set(x)

    deg32 = _deg_sc(dst, w, n_pad, nb).reshape(NW, n_pad)
    dinv_row = _dinv_tc(deg32)
    dinv = dinv_row.reshape(n_pad)
    dinv_col = dinv.reshape(n_pad, 1)

    blk = 1024
    p0 = _matmul_tc(x_p, params["W0"], blk)

    def layer(p_cur, i, w_next, split):
        psp = p_cur.reshape(split * n_pad, h // split)
        acc = _spmm_sc(src, dst, w, dinv, psp, h, n_pad, nbw, split)
        return _mid_tc(
            acc, p_cur, dinv_col,
            params["b%d" % i].reshape(1, h),
            params["g%d" % i].reshape(1, h),
            params["be%d" % i].reshape(1, h),
            params["rm%d" % i].reshape(1, h),
            params["rv%d" % i].reshape(1, h),
            w_next, blk, split,
        )

    p1 = layer(p0, 0, params["W1"], 4)
    p2 = layer(p1, 1, params["W2"], 4)
    h3 = layer(p2, 2, None, 4)

    posf = pos.reshape(NW * (s * m // (NW * EB)), EB).astype(I32)
    emb_sum = _pool_sc(h3, posf, s, m, h)

    h2 = params["Wc1"].shape[1]
    c = params["Wc2"].shape[1]
    return _mlp_tc(
        emb_sum, 1.0 / m,
        params["Wc1"], params["bc1"].reshape(1, h2),
        params["Wc2"], params["bc2"].reshape(1, c),
        blk=1024,
    )
